# Initial kernel scaffold; baseline (speedup 1.0000x reference)
import jax
import jax.numpy as jnp
from jax.experimental import pallas as pl


def kernel(x):
    raise NotImplementedError("write your pallas kernel here")


# speedup vs baseline: 1.6217x; 1.6217x over previous
import functools

import jax
import jax.numpy as jnp
from jax import lax
from jax.experimental import pallas as pl
from jax.experimental.pallas import tpu as pltpu
from jax.experimental.pallas import tpu_sc as plsc

R = 128
N = 32768
L = 16
NBITS = 8
D = 1 << NBITS
NPASS = 32 // NBITS
NC, NS = 2, 16
P_TOP = 0.9


def _digit(t, shift):
    return lax.shift_right_logical(t, jnp.int32(shift)) & jnp.int32(D - 1)


def _make_body(rows, n):
    nv = n // L
    chunk = n // L
    chunk_bits = chunk.bit_length() - 1
    rows_per_w = rows // (NC * NS)

    def _sc_body(x_hbm, vals_hbm, order_hbm, bufA, bufB, keysT, counters):
        lane = lax.broadcasted_iota(jnp.int32, (L,), 0)
        ones = jnp.ones((L,), jnp.int32)
        wid = lax.axis_index("s") * NC + lax.axis_index("c")

        def kslot(idx):
            return idx + lax.shift_right_logical(idx, jnp.int32(chunk_bits))

        def row_body(j, _):
            r = wid * rows_per_w + j
            pltpu.sync_copy(x_hbm.at[r], bufA)

            def tr_body(i, mx):
                v = bufA[pl.ds(i * L, L)]
                u = plsc.bitcast(v, jnp.int32)
                t = jnp.where(u < 0, u, ~u & jnp.int32(0x7FFFFFFF))
                e = i * L + lane
                plsc.store_scatter(keysT, [kslot(e)], t)
                return jnp.maximum(mx, v)

            mx16 = lax.fori_loop(0, nv, tr_body,
                                 jnp.full((L,), -jnp.inf, jnp.float32))
            mx = jnp.max(mx16)

            def sum_body(i, acc):
                v = bufA[pl.ds(i * L, L)]
                return acc + jnp.exp(v - mx)

            acc16 = lax.fori_loop(0, nv, sum_body, jnp.zeros((L,), jnp.float32))
            total = jnp.sum(acc16)

            for p in range(NPASS):
                shift = p * NBITS
                last = p == NPASS - 1

                def zero_body(i, _):
                    counters[pl.ds(i * L, L)] = jnp.zeros((L,), jnp.int32)
                    return 0

                lax.fori_loop(0, D, zero_body, 0)

                if p == 0:
                    def hist_body(i, _, shift=shift):
                        t = plsc.load_gather(keysT, [lane * (chunk + 1) + i])
                        d = _digit(t, shift)
                        plsc.addupdate_scatter(counters, [d * L + lane], ones)
                        return 0
                else:
                    src = bufA if p % 2 == 1 else bufB

                    def hist_body(i, _, src=src, shift=shift):
                        iv = src[pl.ds(i * L, L)]
                        idxv = (plsc.bitcast(iv, jnp.int32)
                                if src is bufA else iv)
                        t = plsc.load_gather(keysT, [kslot(idxv)])
                        d = _digit(t, shift)
                        plsc.addupdate_scatter(counters, [d * L + lane], ones)
                        return 0

                lax.fori_loop(0, nv, hist_body, 0)

                def scan_body(i, carry):
                    v = counters[pl.ds(i * L, L)]
                    inc = plsc.cumsum(v)
                    counters[pl.ds(i * L, L)] = inc - v + carry
                    return carry + jnp.max(inc)

                lax.fori_loop(0, D, scan_body, jnp.int32(0))

                def perm_body(i, _, p=p, shift=shift, last=last):
                    if p == 0:
                        t = plsc.load_gather(keysT, [lane * (chunk + 1) + i])
                        idxv = lane * chunk + i
                    else:
                        src = bufA if p % 2 == 1 else bufB
                        iv = src[pl.ds(i * L, L)]
                        idxv = (plsc.bitcast(iv, jnp.int32)
                                if src is bufA else iv)
                        t = plsc.load_gather(keysT, [kslot(idxv)])
                    d = _digit(t, shift)
                    addr = d * L + lane
                    base = plsc.load_gather(counters, [addr])
                    plsc.store_scatter(counters, [addr], base + ones)
                    if last:
                        s = base
                    else:
                        s = ((base & jnp.int32(chunk - 1)) << 4) | \
                            lax.shift_right_logical(base, jnp.int32(chunk_bits))
                    if p % 2 == 0:
                        plsc.store_scatter(bufA, [s],
                                           plsc.bitcast(idxv, jnp.float32))
                    else:
                        plsc.store_scatter(bufB, [s], idxv)
                    return 0

                lax.fori_loop(0, nv, perm_body, 0)

            thresh = jnp.float32(P_TOP) * total

            def mask_body(i, cum):
                idxv = bufB[pl.ds(i * L, L)]
                t = plsc.load_gather(keysT, [kslot(idxv)])
                u = jnp.where(t < 0, t, ~t & jnp.int32(0x7FFFFFFF))
                v = plsc.bitcast(u, jnp.float32)
                e = jnp.exp(v - mx)
                inc = plsc.cumsum(e)
                excl = inc - e + cum
                out = jnp.where(excl < thresh, v, -jnp.inf)
                bufA[pl.ds(i * L, L)] = out
                return cum + jnp.max(inc)

            lax.fori_loop(0, nv, mask_body, jnp.float32(0.0))

            pltpu.sync_copy(bufA, vals_hbm.at[r])
            pltpu.sync_copy(bufB, order_hbm.at[r])
            return 0

        lax.fori_loop(0, rows_per_w, row_body, 0)

    return _sc_body


def _make_kernel(rows, n, interpret=False):
    mesh = plsc.VectorSubcoreMesh(core_axis_name="c", subcore_axis_name="s",
                                  num_cores=NC, num_subcores=NS)
    return pl.kernel(
        _make_body(rows, n),
        out_type=(
            jax.ShapeDtypeStruct((rows, n), jnp.float32),
            jax.ShapeDtypeStruct((rows, n), jnp.int32),
        ),
        mesh=mesh,
        scratch_types=[
            pltpu.VMEM((n,), jnp.float32),
            pltpu.VMEM((n,), jnp.int32),
            pltpu.VMEM((n + L,), jnp.int32),
            pltpu.VMEM((D * L,), jnp.int32),
        ],
        compiler_params=pltpu.CompilerParams(needs_layout_passes=False),
        interpret=interpret,
    )


@jax.jit
def kernel(x):
    return _make_kernel(R, N)(x)


# speedup vs baseline: 2.3480x; 1.4479x over previous
import functools

import jax
import jax.numpy as jnp
from jax import lax
from jax.experimental import pallas as pl
from jax.experimental.pallas import tpu as pltpu
from jax.experimental.pallas import tpu_sc as plsc

R = 128
N = 32768
L = 16
NBITS = 8
D = 1 << NBITS
NPASS = 32 // NBITS
NC, NS = 2, 16
PG = 8
P_TOP = 0.9


def _digit(t, shift):
    return lax.shift_right_logical(t, jnp.int32(shift)) & jnp.int32(D - 1)


def _make_body(rows, n):
    nv = n // L
    chunk = n // L
    chunk_bits = chunk.bit_length() - 1
    rows_per_w = rows // (NC * NS)

    def _sc_body(x_hbm, vals_hbm, order_hbm, bufA, bufB, keysT, counters):
        lane = lax.broadcasted_iota(jnp.int32, (L,), 0)
        ones = jnp.ones((L,), jnp.int32)
        wid = lax.axis_index("s") * NC + lax.axis_index("c")

        def kslot(idx):
            return idx + lax.shift_right_logical(idx, jnp.int32(chunk_bits))

        def row_body(j, _):
            r = wid * rows_per_w + j
            pltpu.sync_copy(x_hbm.at[r], bufA)

            def tr_body(i, mx):
                v = bufA[pl.ds(i * L, L)]
                u = plsc.bitcast(v, jnp.int32)
                t = jnp.where(u < 0, u, ~u & jnp.int32(0x7FFFFFFF))
                e = i * L + lane
                plsc.store_scatter(keysT, [kslot(e)], t)
                return jnp.maximum(mx, v)

            mx16 = lax.fori_loop(0, nv, tr_body,
                                 jnp.full((L,), -jnp.inf, jnp.float32))
            mx = jnp.max(mx16)

            def sum_body(i, acc):
                v = bufA[pl.ds(i * L, L)]
                return acc + jnp.exp(v - mx)

            acc16 = lax.fori_loop(0, nv, sum_body, jnp.zeros((L,), jnp.float32))
            total = jnp.sum(acc16)

            for p in range(NPASS):
                shift = p * NBITS
                last = p == NPASS - 1

                def zero_body(i, _):
                    counters[pl.ds(i * L, L)] = jnp.zeros((L,), jnp.int32)
                    return 0

                lax.fori_loop(0, D, zero_body, 0)

                if p == 0:
                    def hist_body(i, _, shift=shift):
                        t = plsc.load_gather(keysT, [lane * (chunk + 1) + i])
                        d = _digit(t, shift)
                        plsc.addupdate_scatter(counters, [d * L + lane], ones)
                        return 0
                else:
                    src = bufA if p % 2 == 1 else bufB

                    def hist_body(i, _, src=src, shift=shift):
                        iv = src[pl.ds(i * L, L)]
                        idxv = (plsc.bitcast(iv, jnp.int32)
                                if src is bufA else iv)
                        t = plsc.load_gather(keysT, [kslot(idxv)])
                        d = _digit(t, shift)
                        plsc.addupdate_scatter(counters, [d * L + lane], ones)
                        return 0

                lax.fori_loop(0, nv, hist_body, 0, unroll=4)

                def scan_body(i, carry):
                    v = counters[pl.ds(i * L, L)]
                    inc = plsc.cumsum(v)
                    counters[pl.ds(i * L, L)] = inc - v + carry
                    return carry + jnp.max(inc)

                lax.fori_loop(0, D, scan_body, jnp.int32(0))

                def perm_group(g, _, p=p, shift=shift, last=last):
                    ds_, idxs = [], []
                    for k in range(PG):
                        i = g * PG + k
                        if p == 0:
                            t = plsc.load_gather(
                                keysT, [lane * (chunk + 1) + i])
                            idxv = lane * chunk + i
                        else:
                            src = bufA if p % 2 == 1 else bufB
                            iv = src[pl.ds(i * L, L)]
                            idxv = (plsc.bitcast(iv, jnp.int32)
                                    if src is bufA else iv)
                            t = plsc.load_gather(keysT, [kslot(idxv)])
                        ds_.append(_digit(t, shift))
                        idxs.append(idxv)
                    bases = [plsc.load_gather(counters, [d * L + lane])
                             for d in ds_]
                    zero = jnp.zeros((L,), jnp.int32)
                    for k in range(PG):
                        occ = zero
                        for j in range(k):
                            occ = occ + jnp.where(ds_[j] == ds_[k], 1, 0)
                        pos = bases[k] + occ
                        is_last = jnp.full((L,), True)
                        for j in range(k + 1, PG):
                            is_last = is_last & (ds_[j] != ds_[k])
                        plsc.store_scatter(counters, [ds_[k] * L + lane],
                                           pos + ones, mask=is_last)
                        if last:
                            s = pos
                        else:
                            s = ((pos & jnp.int32(chunk - 1)) << 4) | \
                                lax.shift_right_logical(
                                    pos, jnp.int32(chunk_bits))
                        if p % 2 == 0:
                            plsc.store_scatter(bufA, [s],
                                               plsc.bitcast(idxs[k],
                                                            jnp.float32))
                        else:
                            plsc.store_scatter(bufB, [s], idxs[k])
                    return 0

                lax.fori_loop(0, nv // PG, perm_group, 0)

            thresh = jnp.float32(P_TOP) * total

            def mask_body(i, cum):
                idxv = bufB[pl.ds(i * L, L)]
                t = plsc.load_gather(keysT, [kslot(idxv)])
                u = jnp.where(t < 0, t, ~t & jnp.int32(0x7FFFFFFF))
                v = plsc.bitcast(u, jnp.float32)
                e = jnp.exp(v - mx)
                inc = plsc.cumsum(e)
                excl = inc - e + cum
                out = jnp.where(excl < thresh, v, -jnp.inf)
                bufA[pl.ds(i * L, L)] = out
                return cum + jnp.max(inc)

            lax.fori_loop(0, nv, mask_body, jnp.float32(0.0))

            pltpu.sync_copy(bufA, vals_hbm.at[r])
            pltpu.sync_copy(bufB, order_hbm.at[r])
            return 0

        lax.fori_loop(0, rows_per_w, row_body, 0)

    return _sc_body


def _make_kernel(rows, n, interpret=False):
    mesh = plsc.VectorSubcoreMesh(core_axis_name="c", subcore_axis_name="s",
                                  num_cores=NC, num_subcores=NS)
    return pl.kernel(
        _make_body(rows, n),
        out_type=(
            jax.ShapeDtypeStruct((rows, n), jnp.float32),
            jax.ShapeDtypeStruct((rows, n), jnp.int32),
        ),
        mesh=mesh,
        scratch_types=[
            pltpu.VMEM((n,), jnp.float32),
            pltpu.VMEM((n,), jnp.int32),
            pltpu.VMEM((n + L,), jnp.int32),
            pltpu.VMEM((D * L,), jnp.int32),
        ],
        compiler_params=pltpu.CompilerParams(needs_layout_passes=False),
        interpret=interpret,
    )


@jax.jit
def kernel(x):
    return _make_kernel(R, N)(x)
